# baseline (speedup 1.0000x reference)
import jax
import jax.numpy as jnp
from jax.experimental import pallas as pl


def _i32(x):
    return jnp.asarray(x, dtype=jnp.int32)


def _criti_kernel(rn_ref, ori_ref, emb_ref, out_ref):
    H = rn_ref.shape[0] + 1
    KTP = rn_ref.shape[2]
    TP = ori_ref.shape[1]
    K = KTP // TP
    n_cand = H * K
    ori1 = ori_ref[...]
    em1 = emb_ref[...]
    ori = jnp.concatenate([ori1] * K, axis=1)
    em = jnp.concatenate([em1] * K, axis=1)
    out_ref[:, n_cand * TP:n_cand * TP + TP] = (1.0 - 1.0 / H) * em1
    neg_base = (n_cand + 1) * TP
    pos0 = jnp.float32(1.0 - 2.0 / H) * em
    neg0 = jnp.where(ori == 1, pos0, -pos0)
    out_ref[:, 0:KTP] = pos0
    out_ref[:, neg_base:neg_base + KTP] = neg0
    for h in range(1, H):
        p1 = jnp.float32(1.0 - (h + 1) / H)
        p2 = jnp.float32(1.0 - (h + 2) / H)
        lowmask = (1 << h) - 1
        flip = 1 << h
        x = rn_ref[h - 1].astype(jnp.int32)
        m = x & lowmask
        scale = jnp.where(m == lowmask, p2, p1)
        pos = scale * em
        neg = jnp.where(ori == (m | flip), pos, -pos)
        out_ref[:, h * KTP:(h + 1) * KTP] = pos
        out_ref[:, neg_base + h * KTP:neg_base + (h + 1) * KTP] = neg


def kernel(ori_int, random_numbers, emb):
    H, B, K, TP = random_numbers.shape
    NC = 2 * H * K + 1
    rn16 = random_numbers[1:].reshape(-1).astype(jnp.uint16).reshape(H - 1, B, K * TP)
    ori32 = ori_int.astype(jnp.int32)
    Bb = 128
    out_flat = pl.pallas_call(
        _criti_kernel,
        grid=(B // Bb,),
        in_specs=[
            pl.BlockSpec((H - 1, Bb, K * TP), lambda i: (_i32(0), i, _i32(0))),
            pl.BlockSpec((Bb, TP), lambda i: (i, _i32(0))),
            pl.BlockSpec((Bb, TP), lambda i: (i, _i32(0))),
        ],
        out_specs=pl.BlockSpec((Bb, NC * TP), lambda i: (i, _i32(0))),
        out_shape=jax.ShapeDtypeStruct((B, NC * TP), jnp.float32),
    )(rn16, ori32, emb)
    return out_flat.reshape(B, NC, TP)


# speedup vs baseline: 1.0083x; 1.0083x over previous
import jax
import jax.numpy as jnp
from jax.experimental import pallas as pl


def _i32(x):
    return jnp.asarray(x, dtype=jnp.int32)


def _criti_kernel(rn_ref, ori_ref, emb_ref, out_ref):
    H = rn_ref.shape[0] + 1
    KTP = rn_ref.shape[2]
    TP = ori_ref.shape[1]
    K = KTP // TP
    n_cand = H * K
    ori1 = ori_ref[...]
    em1 = emb_ref[...]
    ori = jnp.concatenate([ori1] * K, axis=1)
    em = jnp.concatenate([em1] * K, axis=1)
    out_ref[:, n_cand * TP:n_cand * TP + TP] = (1.0 - 1.0 / H) * em1
    neg_base = (n_cand + 1) * TP
    pos0 = jnp.float32(1.0 - 2.0 / H) * em
    neg0 = jnp.where(ori == 1, pos0, -pos0)
    out_ref[:, 0:KTP] = pos0
    out_ref[:, neg_base:neg_base + KTP] = neg0
    for h in range(1, H):
        p1 = jnp.float32(1.0 - (h + 1) / H)
        p2 = jnp.float32(1.0 - (h + 2) / H)
        lowmask = (1 << h) - 1
        flip = 1 << h
        x = rn_ref[h - 1].astype(jnp.int32)
        m = x & lowmask
        scale = jnp.where(m == lowmask, p2, p1)
        pos = scale * em
        neg = jnp.where(ori == (m | flip), pos, -pos)
        out_ref[:, h * KTP:(h + 1) * KTP] = pos
        out_ref[:, neg_base + h * KTP:neg_base + (h + 1) * KTP] = neg


def kernel(ori_int, random_numbers, emb):
    H, B, K, TP = random_numbers.shape
    NC = 2 * H * K + 1
    rn16 = random_numbers[1:].astype(jnp.uint16).reshape(H - 1, B, K * TP)
    ori32 = ori_int.astype(jnp.int32)
    Bb = 256
    out_flat = pl.pallas_call(
        _criti_kernel,
        grid=(B // Bb,),
        in_specs=[
            pl.BlockSpec((H - 1, Bb, K * TP), lambda i: (_i32(0), i, _i32(0))),
            pl.BlockSpec((Bb, TP), lambda i: (i, _i32(0))),
            pl.BlockSpec((Bb, TP), lambda i: (i, _i32(0))),
        ],
        out_specs=pl.BlockSpec((Bb, NC * TP), lambda i: (i, _i32(0))),
        out_shape=jax.ShapeDtypeStruct((B, NC * TP), jnp.float32),
    )(rn16, ori32, emb)
    return out_flat.reshape(B, NC, TP)


# speedup vs baseline: 1.0094x; 1.0010x over previous
import jax
import jax.numpy as jnp
from jax.experimental import pallas as pl


def _i32(x):
    return jnp.asarray(x, dtype=jnp.int32)


def _criti_kernel(rn_ref, ori_ref, emb_ref, out_ref):
    H = rn_ref.shape[0] + 1
    KTP = rn_ref.shape[2]
    TP = ori_ref.shape[1]
    K = KTP // TP
    n_cand = H * K
    ori1 = ori_ref[...]
    em1 = emb_ref[...]
    ori = jnp.concatenate([ori1] * K, axis=1)
    em = jnp.concatenate([em1] * K, axis=1)
    out_ref[:, n_cand * TP:n_cand * TP + TP] = (1.0 - 1.0 / H) * em1
    neg_base = (n_cand + 1) * TP
    pos0 = jnp.float32(1.0 - 2.0 / H) * em
    neg0 = jnp.where(ori == 1, pos0, -pos0)
    out_ref[:, 0:KTP] = pos0
    out_ref[:, neg_base:neg_base + KTP] = neg0
    for h in range(1, H):
        p1 = jnp.float32(1.0 - (h + 1) / H)
        p2 = jnp.float32(1.0 - (h + 2) / H)
        lowmask = (1 << h) - 1
        flip = 1 << h
        x = rn_ref[h - 1].astype(jnp.int32)
        m = x & lowmask
        scale = jnp.where(m == lowmask, p2, p1)
        pos = scale * em
        neg = jnp.where(ori == (m | flip), pos, -pos)
        out_ref[:, h * KTP:(h + 1) * KTP] = pos
        out_ref[:, neg_base + h * KTP:neg_base + (h + 1) * KTP] = neg


def kernel(ori_int, random_numbers, emb):
    H, B, K, TP = random_numbers.shape
    NC = 2 * H * K + 1
    rn16 = random_numbers[1:].astype(jnp.uint16).reshape(H - 1, B, K * TP)
    ori32 = ori_int.astype(jnp.int32)
    Bb = 512
    out_flat = pl.pallas_call(
        _criti_kernel,
        grid=(B // Bb,),
        in_specs=[
            pl.BlockSpec((H - 1, Bb, K * TP), lambda i: (_i32(0), i, _i32(0))),
            pl.BlockSpec((Bb, TP), lambda i: (i, _i32(0))),
            pl.BlockSpec((Bb, TP), lambda i: (i, _i32(0))),
        ],
        out_specs=pl.BlockSpec((Bb, NC * TP), lambda i: (i, _i32(0))),
        out_shape=jax.ShapeDtypeStruct((B, NC * TP), jnp.float32),
    )(rn16, ori32, emb)
    return out_flat.reshape(B, NC, TP)
